# baseline (speedup 1.0000x reference)
import functools

import jax
import jax.numpy as jnp
from jax import lax
from jax.experimental import pallas as pl
from jax.experimental.pallas import tpu as pltpu
from jax.experimental.pallas import tpu_sc as plsc

_NC = 2
_NS = 16
_NW = _NC * _NS
_L = 16

_CHUNK = 1024
_SUPER = 10


def _embed_body(idx_hbm, table_hbm, out_hbm, table_v, idx_super,
                rows0, rows1, wsem):
    VD = table_hbm.shape[0]
    B = idx_hbm.shape[0]
    D = out_hbm.shape[0] // B
    per_w = B // _NW
    chunks_per_w = per_w // _CHUNK
    n_super = chunks_per_w // _SUPER

    wid = lax.axis_index("s") * _NC + lax.axis_index("c")
    base = wid * per_w

    pltpu.sync_copy(table_hbm, table_v)

    rowsb = (rows0, rows1)
    iota_d = lax.iota(jnp.int32, _L) * D

    def gather_chunk(coff, rows):
        def group(g, carry):
            iv = idx_super[pl.ds(coff + g * _L, _L)]
            ld = iv * D
            st = iota_d + g * (_L * D)
            for d in range(D):
                val = plsc.load_gather(table_v, [ld + d])
                plsc.store_scatter(rows, [st + d], val)
            return carry

        lax.fori_loop(0, _CHUNK // _L, group, 0, unroll=False)

    def super_chunk(s, carry):
        sbase = base + s * _SUPER * _CHUNK
        pltpu.sync_copy(idx_hbm.at[pl.ds(sbase, _SUPER * _CHUNK)], idx_super)

        def wb_start(c):
            return pltpu.async_copy(
                rowsb[c % 2],
                out_hbm.at[pl.ds((sbase + c * _CHUNK) * D, _CHUNK * D)],
                wsem)

        wh = None
        for c in range(_SUPER):
            gather_chunk(c * _CHUNK, rowsb[c % 2])
            if wh is not None:
                wh.wait()
            wh = wb_start(c)
        wh.wait()
        return carry

    lax.fori_loop(0, n_super, super_chunk, 0)


def kernel(indices, table):
    R, C = indices.shape
    V, D = table.shape
    B = R * C
    assert B % (_NW * _CHUNK * _SUPER) == 0

    idx_flat = indices.reshape(B)
    table_flat = table.reshape(V * D)

    mesh = plsc.VectorSubcoreMesh(core_axis_name="c", subcore_axis_name="s")
    embed = functools.partial(
        pl.kernel,
        out_type=jax.ShapeDtypeStruct((B * D,), jnp.float32),
        mesh=mesh,
        scratch_types=[
            pltpu.VMEM((V * D,), jnp.float32),
            pltpu.VMEM((_SUPER * _CHUNK,), jnp.int32),
            pltpu.VMEM((_CHUNK * D,), jnp.float32),
            pltpu.VMEM((_CHUNK * D,), jnp.float32),
            pltpu.SemaphoreType.DMA,
        ],
        compiler_params=pltpu.CompilerParams(use_tc_tiling_on_sc=False,
                                             needs_layout_passes=False),
    )(_embed_body)

    out = embed(idx_flat, table_flat)
    return out.reshape(R, C, D)


# speedup vs baseline: 1.2260x; 1.2260x over previous
import functools

import jax
import jax.numpy as jnp
from jax import lax
from jax.experimental import pallas as pl
from jax.experimental.pallas import tpu as pltpu
from jax.experimental.pallas import tpu_sc as plsc

_NC = 2
_NS = 16
_NW = _NC * _NS
_L = 16

_CHUNK = 512
_SUPER = 4


def _embed_body(idx_hbm, table_hbm, out_hbm, table_v, idx_super,
                rows0, rows1, wsem):
    VD = table_hbm.shape[0]
    B = idx_hbm.shape[0]
    D = out_hbm.shape[0] // B
    per_w = B // _NW
    chunks_per_w = per_w // _CHUNK
    n_super = chunks_per_w // _SUPER

    wid = lax.axis_index("s") * _NC + lax.axis_index("c")
    base = wid * per_w

    pltpu.sync_copy(table_hbm, table_v)

    rowsb = (rows0, rows1)
    iota_d = lax.iota(jnp.int32, _L) * D

    def gather_chunk(coff, rows):
        @plsc.parallel_loop(0, _CHUNK // _L, unroll=1)
        def group(g):
            iv = idx_super[pl.ds(coff + g * _L, _L)]
            ld = iv * D
            st = iota_d + g * (_L * D)
            for d in range(D):
                val = plsc.load_gather(table_v, [ld + d])
                plsc.store_scatter(rows, [st + d], val)

    def super_chunk(s, carry):
        sbase = base + s * _SUPER * _CHUNK
        pltpu.sync_copy(idx_hbm.at[pl.ds(sbase, _SUPER * _CHUNK)], idx_super)

        def wb_start(c):
            return pltpu.async_copy(
                rowsb[c % 2],
                out_hbm.at[pl.ds((sbase + c * _CHUNK) * D, _CHUNK * D)],
                wsem)

        wh = None
        for c in range(_SUPER):
            gather_chunk(c * _CHUNK, rowsb[c % 2])
            if wh is not None:
                wh.wait()
            wh = wb_start(c)
        wh.wait()
        return carry

    lax.fori_loop(0, n_super, super_chunk, 0)


def kernel(indices, table):
    R, C = indices.shape
    V, D = table.shape
    B = R * C
    assert B % (_NW * _CHUNK * _SUPER) == 0

    idx_flat = indices.reshape(B)
    table_flat = table.reshape(V * D)

    mesh = plsc.VectorSubcoreMesh(core_axis_name="c", subcore_axis_name="s")
    embed = functools.partial(
        pl.kernel,
        out_type=jax.ShapeDtypeStruct((B * D,), jnp.float32),
        mesh=mesh,
        scratch_types=[
            pltpu.VMEM((V * D,), jnp.float32),
            pltpu.VMEM((_SUPER * _CHUNK,), jnp.int32),
            pltpu.VMEM((_CHUNK * D,), jnp.float32),
            pltpu.VMEM((_CHUNK * D,), jnp.float32),
            pltpu.SemaphoreType.DMA,
        ],
        compiler_params=pltpu.CompilerParams(use_tc_tiling_on_sc=False,
                                             needs_layout_passes=False),
    )(_embed_body)

    out = embed(idx_flat, table_flat)
    return out.reshape(R, C, D)
